# baseline (speedup 1.0000x reference)
import functools

import jax
import jax.numpy as jnp
from jax.experimental import pallas as pl

_B = 16384
_LOC = 64
_K = 64
_EPS = 1e-08
_BLOCK = 2048
_R = _BLOCK // 128

_HI = jax.lax.Precision.HIGHEST


def _fused_body(fu_ref, fl_ref, fp_ref, w1t_ref, b1_ref, w2t_ref, b2_ref,
                w3t_ref, b3_ref, tcat_ref, bcat_ref, ut_ref, out_ref):
    f32 = jnp.float32
    h = jnp.dot(fu_ref[...], w1t_ref[...], preferred_element_type=f32,
                precision=_HI) + b1_ref[...]
    h = jnp.where(h >= 0, h, 0.01 * h)
    h = jnp.dot(h, w2t_ref[...], preferred_element_type=f32,
                precision=_HI) + b2_ref[...]
    h = jnp.where(h >= 0, h, 0.01 * h)
    z = jnp.dot(h, w3t_ref[...], preferred_element_type=f32,
                precision=_HI) + b3_ref[...]

    zmax = jnp.max(z, axis=1, keepdims=True)
    m = (z == zmax).astype(f32)
    cnt = jnp.dot(m, ut_ref[...], preferred_element_type=f32)
    onehot = m * (cnt == 1.0).astype(f32)

    fl = fl_ref[...]
    a_cat = jnp.dot(fl, tcat_ref[...], preferred_element_type=f32,
                    precision=_HI) + bcat_ref[...]
    o2 = jnp.concatenate([onehot, onehot], axis=1)
    s = o2 * a_cat
    lane = jax.lax.broadcasted_iota(jnp.int32, s.shape, 1)
    mu = jnp.sum(jnp.where(lane < _K, s, 0.0), axis=1, keepdims=True)
    d0 = jnp.sum(jnp.where(lane >= _K, s, 0.0), axis=1, keepdims=True)

    delta = jnp.abs(d0) + _EPS
    diff = mu - fp_ref[...]
    pd = (diff * diff) / (delta * delta) * 0.5
    terms = pd - jnp.log(delta)
    part = jnp.sum(terms, axis=(0, 1), keepdims=True)[:1, :1]

    @pl.when(pl.program_id(0) == 0)
    def _init():
        out_ref[...] = jnp.zeros((1, 1), jnp.float32)

    out_ref[...] += part


@functools.partial(jax.jit, static_argnames=())
def kernel(feat_user, feat_loc, feat_price, W1, b1, W2, b2, W3, b3, theta):
    n_blocks = _B // _BLOCK
    w1t = W1.T
    w2t = W2.T
    w3t = W3.T
    tcat = jnp.concatenate([theta[:, 0, :_LOC].T,
                            theta[:, 1, :_LOC].T], axis=1)
    bcat = jnp.concatenate([theta[:, 0, _LOC],
                            theta[:, 1, _LOC]]).reshape(1, 2 * _K)
    ut = jnp.triu(jnp.ones((_K, _K), jnp.float32))
    b1r = b1.reshape(1, -1)
    b2r = b2.reshape(1, -1)
    b3r = b3.reshape(1, -1)

    full = lambda shape: pl.BlockSpec(shape, lambda i: tuple(0 for _ in shape))
    grid_spec = pl.GridSpec(
        grid=(n_blocks,),
        in_specs=[
            pl.BlockSpec((_BLOCK, 128), lambda i: (i, 0)),
            pl.BlockSpec((_BLOCK, _LOC), lambda i: (i, 0)),
            pl.BlockSpec((_BLOCK, 1), lambda i: (i, 0)),
            full((128, 32)), full((1, 32)),
            full((32, 16)), full((1, 16)),
            full((16, _K)), full((1, _K)),
            full((_K, 2 * _K)), full((1, 2 * _K)),
            full((_K, _K)),
        ],
        out_specs=pl.BlockSpec((1, 1), lambda i: (0, 0)),
    )
    acc = pl.pallas_call(
        _fused_body,
        grid_spec=grid_spec,
        out_shape=jax.ShapeDtypeStruct((1, 1), jnp.float32),
    )(feat_user, feat_loc, feat_price, w1t, b1r, w2t, b2r, w3t, b3r, tcat, bcat, ut)
    return acc[0, 0] / _B
